# baseline (speedup 1.0000x reference)
import functools

import jax
import jax.numpy as jnp
from jax import lax
from jax.experimental import pallas as pl
from jax.experimental.pallas import tpu as pltpu
from jax.experimental.pallas import tpu_sc as plsc

N = 10000
E = 320000
D = 128
NC = 2
NS = 16
NW = NC * NS
EW = E // NW
KB = 125
NIT = EW // KB
NH = 2
NIH = NIT // NH
NP = 10240
RPT = NP // NS

_mesh = plsc.VectorSubcoreMesh(
    core_axis_name="c", subcore_axis_name="s", num_cores=NC, num_subcores=NS
)


@functools.partial(
    pl.kernel,
    out_type=jax.ShapeDtypeStruct((3, NC, NP, D), jnp.float32),
    mesh=_mesh,
    scratch_types=[
        pltpu.VMEM((NIH, KB), jnp.int32),
        pltpu.VMEM((NIH, KB), jnp.int32),
        pltpu.VMEM((KB, D), jnp.float32),
        pltpu.VMEM((KB, D), jnp.float32),
        pltpu.VMEM_SHARED((NP, D), jnp.float32),
        pltpu.SemaphoreType.DMA,
        pltpu.SemaphoreType.DMA,
        pltpu.SemaphoreType.DMA,
        pltpu.SemaphoreType.DMA,
    ],
)
def _spmm3(t0, t1, t2, eidx_hbm, zeros_hbm, out_hbm,
           si_v, di_v, r0, r1,
           acc, sg0, sg1, sc0, sc1):
    cid = lax.axis_index("c")
    sid = lax.axis_index("s")
    wid = cid * NS + sid
    tabs = [t0, t1, t2]
    rows = [r0, r1]
    sg = [sg0, sg1]
    scs = [sc0, sc1]

    for c in range(3):
        tab = tabs[c]

        def fire_g(i, p):
            pltpu.async_copy(tab.at[si_v.at[i]], rows[p], sg[p])

        def fire_s(i, p):
            pltpu.async_copy(rows[p], acc.at[di_v.at[i]], scs[p], add=True)

        def wait_g(p):
            pltpu.make_async_copy(tab.at[si_v.at[0]], rows[p], sg[p]).wait()

        def wait_s(p):
            pltpu.make_async_copy(rows[p], acc.at[di_v.at[0]], scs[p]).wait()

        for h in range(NH):
            pltpu.sync_copy(eidx_hbm.at[1, wid, h], si_v)
            pltpu.sync_copy(eidx_hbm.at[0, wid, h], di_v)

            fire_g(0, 0)
            fire_g(1, 1)

            if c == 0 and h == 0:
                pltpu.sync_copy(zeros_hbm, acc.at[pl.ds(sid * RPT, RPT)])
                plsc.subcore_barrier()

            def pair(j, carry):
                for p in range(2):
                    i = 2 * j + p
                    wait_g(p)
                    fire_s(i, p)

                    @pl.when(i + 2 < NIH)
                    def _():
                        wait_s(p)
                        fire_g(i + 2, p)

                return carry

            lax.fori_loop(0, NIH // 2, pair, 0)
            wait_s(0)
            wait_s(1)

        plsc.subcore_barrier()
        pltpu.sync_copy(acc.at[pl.ds(sid * RPT, RPT)],
                        out_hbm.at[c, cid, pl.ds(sid * RPT, RPT)])
        if c < 2:
            pltpu.sync_copy(zeros_hbm, acc.at[pl.ds(sid * RPT, RPT)])
            plsc.subcore_barrier()


BN = 1000
GRID = N // BN


def _prep_body(x_ref, a_ref, b_ref, xa_ref, xb_ref):
    x = x_ref[...]
    xa_ref[...] = x * a_ref[...]
    xb_ref[...] = x * b_ref[...]


_prep = pl.pallas_call(
    _prep_body,
    grid=(GRID,),
    in_specs=[pl.BlockSpec((BN, D), lambda i: (i, 0)),
              pl.BlockSpec((BN, 1), lambda i: (i, 0)),
              pl.BlockSpec((BN, 1), lambda i: (i, 0))],
    out_specs=[pl.BlockSpec((BN, D), lambda i: (i, 0)),
               pl.BlockSpec((BN, D), lambda i: (i, 0))],
    out_shape=[jax.ShapeDtypeStruct((N, D), jnp.float32)] * 2,
)


def _merge1_body(p_ref, r_ref, h0_ref, h1_ref, h2_ref):
    r = r_ref[...]
    h0_ref[...] = r * (p_ref[0, 0] + p_ref[0, 1])
    h1_ref[...] = r * (p_ref[1, 0] + p_ref[1, 1])
    h2_ref[...] = r * (p_ref[2, 0] + p_ref[2, 1])


_merge1 = pl.pallas_call(
    _merge1_body,
    grid=(GRID,),
    in_specs=[pl.BlockSpec((3, NC, BN, D), lambda i: (0, 0, i, 0)),
              pl.BlockSpec((BN, 1), lambda i: (i, 0))],
    out_specs=[pl.BlockSpec((BN, D), lambda i: (i, 0))] * 3,

    out_shape=[jax.ShapeDtypeStruct((N, D), jnp.float32)] * 3,
)


BNF = 80


def _final_body(t0, t1, t2, h0, h1, h2, p_ref, r_ref, out_ref):
    r = r_ref[...]
    ts = (t0, t1, t2)
    hs = (h0, h1, h2)
    cols = [ts[k][...] for k in range(3)]
    cols += [hs[k][...] for k in range(3)]
    cols += [r * (p_ref[k, 0] + p_ref[k, 1]) - cols[k] for k in range(3)]
    out_ref[...] = jnp.stack(cols, axis=-1)


_final = pl.pallas_call(
    _final_body,
    grid=(N // BNF,),
    in_specs=[pl.BlockSpec((BNF, D), lambda i: (i, 0))] * 6
    + [pl.BlockSpec((3, NC, BNF, D), lambda i: (0, 0, i, 0)),
       pl.BlockSpec((BNF, 1), lambda i: (i, 0))],
    out_specs=pl.BlockSpec((BNF, D, 9), lambda i: (i, 0, 0)),
    out_shape=jax.ShapeDtypeStruct((N, D, 9), jnp.float32),
)


def kernel(x, edge_index, deg):
    eidx = edge_index.reshape(2, NW, NH, NIH, KB)
    a = lax.rsqrt(deg)
    b = jnp.sqrt(deg)
    r = jnp.reciprocal(deg)
    zeros = jnp.zeros((RPT, D), jnp.float32)
    xa, xb = _prep(x, a, b)
    p1 = _spmm3(x, xa, xb, eidx, zeros)
    h0, h1, h2 = _merge1(p1, r)
    p2 = _spmm3(h0, h1, h2, eidx, zeros)
    return _final(x, xa, xb, h0, h1, h2, p2, r)


# speedup vs baseline: 3.5438x; 3.5438x over previous
import functools

import jax
import jax.numpy as jnp
from jax import lax
from jax.experimental import pallas as pl
from jax.experimental.pallas import tpu as pltpu
from jax.experimental.pallas import tpu_sc as plsc

N = 10000
E = 320000
D = 128
NC = 2
NS = 16
NW = NC * NS
EW = E // NW
KB = 125
NIT = EW // KB
NH = 2
NIH = NIT // NH
NP = 10240
RPT = NP // NS

_mesh = plsc.VectorSubcoreMesh(
    core_axis_name="c", subcore_axis_name="s", num_cores=NC, num_subcores=NS
)


@functools.partial(
    pl.kernel,
    out_type=jax.ShapeDtypeStruct((3, NC, NP, D), jnp.float32),
    mesh=_mesh,
    scratch_types=[
        pltpu.VMEM((NIH, KB), jnp.int32),
        pltpu.VMEM((NIH, KB), jnp.int32),
        pltpu.VMEM((KB, D), jnp.float32),
        pltpu.VMEM((KB, D), jnp.float32),
        pltpu.VMEM_SHARED((NP, D), jnp.float32),
        pltpu.SemaphoreType.DMA,
        pltpu.SemaphoreType.DMA,
        pltpu.SemaphoreType.DMA,
        pltpu.SemaphoreType.DMA,
    ],
)
def _spmm3(t0, t1, t2, eidx_hbm, zeros_hbm, out_hbm,
           si_v, di_v, r0, r1,
           acc, sg0, sg1, sc0, sc1):
    cid = lax.axis_index("c")
    sid = lax.axis_index("s")
    wid = cid * NS + sid
    tabs = [t0, t1, t2]
    rows = [r0, r1]
    sg = [sg0, sg1]
    scs = [sc0, sc1]

    for c in range(3):
        tab = tabs[c]

        def fire_g(i, p):
            pltpu.async_copy(tab.at[si_v.at[i]], rows[p], sg[p])

        def fire_s(i, p):
            pltpu.async_copy(rows[p], acc.at[di_v.at[i]], scs[p], add=True)

        def wait_g(p):
            pltpu.make_async_copy(tab.at[si_v.at[0]], rows[p], sg[p]).wait()

        def wait_s(p):
            pltpu.make_async_copy(rows[p], acc.at[di_v.at[0]], scs[p]).wait()

        for h in range(NH):
            pltpu.sync_copy(eidx_hbm.at[1, wid, h], si_v)
            pltpu.sync_copy(eidx_hbm.at[0, wid, h], di_v)

            fire_g(0, 0)
            fire_g(1, 1)

            if c == 0 and h == 0:
                pltpu.sync_copy(zeros_hbm, acc.at[pl.ds(sid * RPT, RPT)])
                plsc.subcore_barrier()

            def pair(j, carry):
                for p in range(2):
                    i = 2 * j + p
                    wait_g(p)
                    fire_s(i, p)

                    @pl.when(i + 2 < NIH)
                    def _():
                        wait_s(p)
                        fire_g(i + 2, p)

                return carry

            lax.fori_loop(0, NIH // 2, pair, 0)
            wait_s(0)
            wait_s(1)

        plsc.subcore_barrier()
        pltpu.sync_copy(acc.at[pl.ds(sid * RPT, RPT)],
                        out_hbm.at[c, cid, pl.ds(sid * RPT, RPT)])
        if c < 2:
            pltpu.sync_copy(zeros_hbm, acc.at[pl.ds(sid * RPT, RPT)])
            plsc.subcore_barrier()


BN = 1000
GRID = N // BN


def _prep_body(x_ref, a_ref, b_ref, xa_ref, xb_ref):
    x = x_ref[...]
    xa_ref[...] = x * a_ref[...]
    xb_ref[...] = x * b_ref[...]


_prep = pl.pallas_call(
    _prep_body,
    grid=(GRID,),
    in_specs=[pl.BlockSpec((BN, D), lambda i: (i, 0)),
              pl.BlockSpec((BN, 1), lambda i: (i, 0)),
              pl.BlockSpec((BN, 1), lambda i: (i, 0))],
    out_specs=[pl.BlockSpec((BN, D), lambda i: (i, 0)),
               pl.BlockSpec((BN, D), lambda i: (i, 0))],
    out_shape=[jax.ShapeDtypeStruct((N, D), jnp.float32)] * 2,
)


def _merge1_body(p_ref, r_ref, h0_ref, h1_ref, h2_ref):
    r = r_ref[...]
    h0_ref[...] = r * (p_ref[0, 0] + p_ref[0, 1])
    h1_ref[...] = r * (p_ref[1, 0] + p_ref[1, 1])
    h2_ref[...] = r * (p_ref[2, 0] + p_ref[2, 1])


_merge1 = pl.pallas_call(
    _merge1_body,
    grid=(GRID,),
    in_specs=[pl.BlockSpec((3, NC, BN, D), lambda i: (0, 0, i, 0)),
              pl.BlockSpec((BN, 1), lambda i: (i, 0))],
    out_specs=[pl.BlockSpec((BN, D), lambda i: (i, 0))] * 3,

    out_shape=[jax.ShapeDtypeStruct((N, D), jnp.float32)] * 3,
)


def _merge2_body(p_ref, r_ref, t0, t1, t2, g0_ref, g1_ref, g2_ref):
    r = r_ref[...]
    g0_ref[...] = r * (p_ref[0, 0] + p_ref[0, 1]) - t0[...]
    g1_ref[...] = r * (p_ref[1, 0] + p_ref[1, 1]) - t1[...]
    g2_ref[...] = r * (p_ref[2, 0] + p_ref[2, 1]) - t2[...]


_merge2 = pl.pallas_call(
    _merge2_body,
    grid=(GRID,),
    in_specs=[pl.BlockSpec((3, NC, BN, D), lambda i: (0, 0, i, 0)),
              pl.BlockSpec((BN, 1), lambda i: (i, 0))]
    + [pl.BlockSpec((BN, D), lambda i: (i, 0))] * 3,
    out_specs=[pl.BlockSpec((BN, D), lambda i: (i, 0))] * 3,
    out_shape=[jax.ShapeDtypeStruct((N, D), jnp.float32)] * 3,
)


def kernel(x, edge_index, deg):
    eidx = edge_index.reshape(2, NW, NH, NIH, KB)
    a = lax.rsqrt(deg)
    b = jnp.sqrt(deg)
    r = jnp.reciprocal(deg)
    zeros = jnp.zeros((RPT, D), jnp.float32)
    xa, xb = _prep(x, a, b)
    p1 = _spmm3(x, xa, xb, eidx, zeros)
    h0, h1, h2 = _merge1(p1, r)
    p2 = _spmm3(h0, h1, h2, eidx, zeros)
    g0, g1, g2 = _merge2(p2, r, x, xa, xb)
    return jnp.stack([x, xa, xb, h0, h1, h2, g0, g1, g2], axis=-1)


# speedup vs baseline: 3.5945x; 1.0143x over previous
import functools

import jax
import jax.numpy as jnp
from jax import lax
from jax.experimental import pallas as pl
from jax.experimental.pallas import tpu as pltpu
from jax.experimental.pallas import tpu_sc as plsc

N = 10000
E = 320000
D = 128
NC = 2
NS = 16
NW = NC * NS
EW = E // NW
KB = 125
NIT = EW // KB
NH = 2
NIH = NIT // NH
NP = 10240
RPT = NP // NS

_mesh = plsc.VectorSubcoreMesh(
    core_axis_name="c", subcore_axis_name="s", num_cores=NC, num_subcores=NS
)


@functools.partial(
    pl.kernel,
    out_type=jax.ShapeDtypeStruct((NC, NP, D), jnp.float32),
    mesh=_mesh,
    scratch_types=[
        pltpu.VMEM((NIH, KB), jnp.int32),
        pltpu.VMEM((NIH, KB), jnp.int32),
        pltpu.VMEM((KB, D), jnp.float32),
        pltpu.VMEM((KB, D), jnp.float32),
        pltpu.VMEM_SHARED((NP, D), jnp.float32),
        pltpu.SemaphoreType.DMA,
        pltpu.SemaphoreType.DMA,
        pltpu.SemaphoreType.DMA,
        pltpu.SemaphoreType.DMA,
    ],
)
def _spmm1(tab, eidx_hbm, zeros_hbm, out_hbm,
           si_v, di_v, r0, r1,
           acc, sg0, sg1, sc0, sc1):
    cid = lax.axis_index("c")
    sid = lax.axis_index("s")
    wid = cid * NS + sid
    rows = [r0, r1]
    sg = [sg0, sg1]
    scs = [sc0, sc1]

    def fire_g(i, p):
        pltpu.async_copy(tab.at[si_v.at[i]], rows[p], sg[p])

    def fire_s(i, p):
        pltpu.async_copy(rows[p], acc.at[di_v.at[i]], scs[p], add=True)

    def wait_g(p):
        pltpu.make_async_copy(tab.at[si_v.at[0]], rows[p], sg[p]).wait()

    def wait_s(p):
        pltpu.make_async_copy(rows[p], acc.at[di_v.at[0]], scs[p]).wait()

    for h in range(NH):
        pltpu.sync_copy(eidx_hbm.at[1, wid, h], si_v)
        pltpu.sync_copy(eidx_hbm.at[0, wid, h], di_v)

        fire_g(0, 0)
        fire_g(1, 1)

        if h == 0:
            pltpu.sync_copy(zeros_hbm, acc.at[pl.ds(sid * RPT, RPT)])
            plsc.subcore_barrier()

        def pair(j, carry):
            for p in range(2):
                i = 2 * j + p
                wait_g(p)
                fire_s(i, p)

                @pl.when(i + 2 < NIH)
                def _():
                    wait_s(p)
                    fire_g(i + 2, p)

            return carry

        lax.fori_loop(0, NIH // 2, pair, 0)
        wait_s(0)
        wait_s(1)

    plsc.subcore_barrier()
    pltpu.sync_copy(acc.at[pl.ds(sid * RPT, RPT)],
                    out_hbm.at[cid, pl.ds(sid * RPT, RPT)])


BN = 1000
GRID = N // BN


def _prep_body(x_ref, a_ref, b_ref, xa_ref, xb_ref):
    x = x_ref[...]
    xa_ref[...] = x * a_ref[...]
    xb_ref[...] = x * b_ref[...]


_prep = pl.pallas_call(
    _prep_body,
    grid=(GRID,),
    in_specs=[pl.BlockSpec((BN, D), lambda i: (i, 0)),
              pl.BlockSpec((BN, 1), lambda i: (i, 0)),
              pl.BlockSpec((BN, 1), lambda i: (i, 0))],
    out_specs=[pl.BlockSpec((BN, D), lambda i: (i, 0)),
               pl.BlockSpec((BN, D), lambda i: (i, 0))],
    out_shape=[jax.ShapeDtypeStruct((N, D), jnp.float32)] * 2,
)


def _merge1_body(p_ref, r_ref, h_ref):
    h_ref[...] = r_ref[...] * (p_ref[0] + p_ref[1])


_merge1 = pl.pallas_call(
    _merge1_body,
    grid=(GRID,),
    in_specs=[pl.BlockSpec((NC, BN, D), lambda i: (0, i, 0)),
              pl.BlockSpec((BN, 1), lambda i: (i, 0))],
    out_specs=pl.BlockSpec((BN, D), lambda i: (i, 0)),
    out_shape=jax.ShapeDtypeStruct((N, D), jnp.float32),
)


def _merge2_body(p_ref, r_ref, t_ref, g_ref):
    g_ref[...] = r_ref[...] * (p_ref[0] + p_ref[1]) - t_ref[...]


_merge2 = pl.pallas_call(
    _merge2_body,
    grid=(GRID,),
    in_specs=[pl.BlockSpec((NC, BN, D), lambda i: (0, i, 0)),
              pl.BlockSpec((BN, 1), lambda i: (i, 0)),
              pl.BlockSpec((BN, D), lambda i: (i, 0))],
    out_specs=pl.BlockSpec((BN, D), lambda i: (i, 0)),
    out_shape=jax.ShapeDtypeStruct((N, D), jnp.float32),
)


def kernel(x, edge_index, deg):
    eidx = edge_index.reshape(2, NW, NH, NIH, KB)
    a = lax.rsqrt(deg)
    b = jnp.sqrt(deg)
    r = jnp.reciprocal(deg)
    zeros = jnp.zeros((RPT, D), jnp.float32)
    xa, xb = _prep(x, a, b)
    q0 = _spmm1(x, eidx, zeros)
    q1 = _spmm1(xa, eidx, zeros)
    q2 = _spmm1(xb, eidx, zeros)
    h0 = _merge1(q0, r)
    h1 = _merge1(q1, r)
    h2 = _merge1(q2, r)
    p0 = _spmm1(h0, eidx, zeros)
    p1 = _spmm1(h1, eidx, zeros)
    p2 = _spmm1(h2, eidx, zeros)
    g0 = _merge2(p0, r, x)
    g1 = _merge2(p1, r, xa)
    g2 = _merge2(p2, r, xb)
    return jnp.stack([x, xa, xb, h0, h1, h2, g0, g1, g2], axis=-1)
